# baseline (speedup 1.0000x reference)
import jax
import jax.numpy as jnp
from jax.experimental import pallas as pl
from jax.experimental.pallas import tpu as pltpu

D_MODEL = 1024
N_EXP = 16
D_EXPERT = 256
N_SHARED = 1
K_FFN = 2
N_ROUTED = N_EXP - N_SHARED
S = 2048
BLK = 256


def _moe_kernel(x_ref, sel_ref, est_ref, k_ref, v_ref,
                out_ref, idx_ref, kscr, vscr, xs_ref, ws_ref, acc_ref):
    t = pl.program_id(0)
    e = pl.program_id(1)

    @pl.when(t == 0)
    def _stash_weights():
        kscr[e] = k_ref[0].astype(jnp.bfloat16)
        vscr[e] = v_ref[0].astype(jnp.bfloat16)

    @pl.when(e == 0)
    def _routing():
        logits = jnp.dot(sel_ref[...], est_ref[...],
                         preferred_element_type=jnp.float32)
        aff = jax.nn.sigmoid(logits)
        ids = jax.lax.broadcasted_iota(jnp.int32, (BLK, N_EXP), 1)
        neg = jnp.where(ids < N_ROUTED, aff, -jnp.inf)
        m1 = jnp.max(neg, axis=1, keepdims=True)
        i1 = jnp.min(jnp.where(neg == m1, ids, N_EXP), axis=1, keepdims=True)
        neg2 = jnp.where(ids == i1, -jnp.inf, neg)
        m2 = jnp.max(neg2, axis=1, keepdims=True)
        i2 = jnp.min(jnp.where(neg2 == m2, ids, N_EXP), axis=1, keepdims=True)
        shared = jnp.full((BLK, 1), N_ROUTED, dtype=jnp.int32)
        idx_ref[...] = jnp.concatenate([i1, i2, shared], axis=1)
        selmask = (ids == i1) | (ids == i2) | (ids >= N_ROUTED)
        ws_ref[...] = jnp.where(selmask, aff, 0.0)
        xs_ref[...] = x_ref[...].astype(jnp.bfloat16)
        acc_ref[...] = jnp.zeros((BLK, D_MODEL), jnp.float32)

    onehot = (jax.lax.broadcasted_iota(jnp.int32, (N_EXP, 1), 0) == e
              ).astype(jnp.float32)
    wcol = jnp.dot(ws_ref[...], onehot,
                   preferred_element_type=jnp.float32)
    h = jnp.dot(xs_ref[...], kscr[e], preferred_element_type=jnp.float32)
    h = h * jax.nn.sigmoid(h)
    hw = (h * wcol).astype(jnp.bfloat16)
    acc_ref[...] += jnp.dot(hw, vscr[e], preferred_element_type=jnp.float32)

    @pl.when(e == N_EXP - 1)
    def _flush():
        out_ref[...] = acc_ref[...]


@jax.jit
def kernel(token_stream, selection_input, keys_w, values_w, expert_sel):
    x = token_stream.reshape(S, D_MODEL)
    sel = selection_input.reshape(S, D_MODEL)
    est = expert_sel.T

    last = N_EXP - 1

    out, sel_idx = pl.pallas_call(
        _moe_kernel,
        grid=(S // BLK, N_EXP),
        in_specs=[
            pl.BlockSpec((BLK, D_MODEL), lambda t, e: (t, 0)),
            pl.BlockSpec((BLK, D_MODEL), lambda t, e: (t, 0)),
            pl.BlockSpec((D_MODEL, N_EXP), lambda t, e: (0, 0)),
            pl.BlockSpec((1, D_MODEL, D_EXPERT),
                         lambda t, e: (jnp.where(t == 0, e, last), 0, 0)),
            pl.BlockSpec((1, D_EXPERT, D_MODEL),
                         lambda t, e: (jnp.where(t == 0, e, last), 0, 0)),
        ],
        out_specs=[
            pl.BlockSpec((BLK, D_MODEL), lambda t, e: (t, 0)),
            pl.BlockSpec((BLK, 3), lambda t, e: (t, 0)),
        ],
        out_shape=[
            jax.ShapeDtypeStruct((S, D_MODEL), jnp.float32),
            jax.ShapeDtypeStruct((S, 3), jnp.int32),
        ],
        scratch_shapes=[
            pltpu.VMEM((N_EXP, D_MODEL, D_EXPERT), jnp.bfloat16),
            pltpu.VMEM((N_EXP, D_EXPERT, D_MODEL), jnp.bfloat16),
            pltpu.VMEM((BLK, D_MODEL), jnp.bfloat16),
            pltpu.VMEM((BLK, N_EXP), jnp.float32),
            pltpu.VMEM((BLK, D_MODEL), jnp.float32),
        ],
        compiler_params=pltpu.CompilerParams(
            dimension_semantics=("arbitrary", "arbitrary"),
        ),
    )(x, sel, est, keys_w, values_w)

    return out.reshape(1, S, D_MODEL), sel_idx.reshape(1, S, 3)


# speedup vs baseline: 2.2136x; 2.2136x over previous
import jax
import jax.numpy as jnp
from jax.experimental import pallas as pl
from jax.experimental.pallas import tpu as pltpu

D_MODEL = 1024
N_EXP = 16
D_EXPERT = 256
N_SHARED = 1
K_FFN = 2
N_ROUTED = N_EXP - N_SHARED
S = 2048
BLK = 256


def _moe_kernel(x_ref, sel_ref, est_ref, k_ref, v_ref, out_ref, idx_ref):
    logits = jnp.dot(sel_ref[...], est_ref[...],
                     preferred_element_type=jnp.float32)
    aff = jax.nn.sigmoid(logits)
    ids = jax.lax.broadcasted_iota(jnp.int32, (BLK, N_EXP), 1)
    neg = jnp.where(ids < N_ROUTED, aff, -jnp.inf)
    m1 = jnp.max(neg, axis=1, keepdims=True)
    i1 = jnp.min(jnp.where(neg == m1, ids, N_EXP), axis=1, keepdims=True)
    neg2 = jnp.where(ids == i1, -jnp.inf, neg)
    m2 = jnp.max(neg2, axis=1, keepdims=True)
    i2 = jnp.min(jnp.where(neg2 == m2, ids, N_EXP), axis=1, keepdims=True)
    shared = jnp.full((BLK, 1), N_ROUTED, dtype=jnp.int32)
    idx_ref[...] = jnp.concatenate([i1, i2, shared], axis=1)
    selmask = (ids == i1) | (ids == i2) | (ids >= N_ROUTED)
    w = jnp.where(selmask, aff, 0.0)

    x = x_ref[...]
    acc = jnp.zeros((BLK, D_MODEL), dtype=jnp.float32)
    for e in range(N_EXP):
        h = jnp.dot(x, k_ref[e], preferred_element_type=jnp.float32)
        h = h * jax.nn.sigmoid(h)
        hw = h * w[:, e:e + 1]
        acc = acc + jnp.dot(hw, v_ref[e], preferred_element_type=jnp.float32)
    out_ref[...] = acc


@jax.jit
def kernel(token_stream, selection_input, keys_w, values_w, expert_sel):
    x = token_stream.reshape(S, D_MODEL)
    sel = selection_input.reshape(S, D_MODEL)
    est = expert_sel.T

    out, sel_idx = pl.pallas_call(
        _moe_kernel,
        grid=(S // BLK,),
        in_specs=[
            pl.BlockSpec((BLK, D_MODEL), lambda t: (t, 0)),
            pl.BlockSpec((BLK, D_MODEL), lambda t: (t, 0)),
            pl.BlockSpec((D_MODEL, N_EXP), lambda t: (0, 0)),
            pl.BlockSpec((N_EXP, D_MODEL, D_EXPERT), lambda t: (0, 0, 0)),
            pl.BlockSpec((N_EXP, D_EXPERT, D_MODEL), lambda t: (0, 0, 0)),
        ],
        out_specs=[
            pl.BlockSpec((BLK, D_MODEL), lambda t: (t, 0)),
            pl.BlockSpec((BLK, 3), lambda t: (t, 0)),
        ],
        out_shape=[
            jax.ShapeDtypeStruct((S, D_MODEL), jnp.float32),
            jax.ShapeDtypeStruct((S, 3), jnp.int32),
        ],
        compiler_params=pltpu.CompilerParams(
            dimension_semantics=("arbitrary",),
        ),
    )(x, sel, est, keys_w, values_w)

    return out.reshape(1, S, D_MODEL), sel_idx.reshape(1, S, 3)
